# initial kernel scaffold (unmeasured)
import numpy as np
import jax
import jax.numpy as jnp
from jax import lax
from jax.experimental import pallas as pl
from jax.experimental.pallas import tpu as pltpu

N_DEV = 16
B, SQ, D = 2, 512, 1024
HQ_LOCAL, DH = 8, 128
SCALE = 0.08838834764831843
ROWS = B * SQ
CHUNK = ROWS // N_DEV
STEPS = N_DEV - 1


def _rope_tables():
    inv = 1.0 / (10000.0 ** (np.arange(0, DH, 2) / DH))
    pos = np.arange(SQ)[:, None] * inv[None, :]
    cos = np.repeat(np.cos(pos), 2, axis=-1).astype(np.float32)
    sin = np.repeat(np.sin(pos), 2, axis=-1).astype(np.float32)
    cos_t = np.tile(cos, (1, HQ_LOCAL))
    sin_t = np.tile(sin, (1, HQ_LOCAL))
    even = np.tile(np.array([1.0, 0.0], np.float32), D // 2)[None, :]
    odd = 1.0 - even
    return cos_t, sin_t, even, odd


def kernel(x, Wq, Wk, Wv, Wo):
    cos_t, sin_t, even, odd = (jnp.asarray(a) for a in _rope_tables())

    def body(x_ref, wq_ref, wk_ref, wv_ref, wo_ref,
             cos_ref, sin_ref, even_ref, odd_ref, out_ref,
             partial_ref, result_ref, rs_buf, ag_buf,
             rs_send, rs_recv, ag_send, ag_recv):
        me = lax.axis_index("i")
        right = lax.rem(me + 1, N_DEV)
        left = lax.rem(me + N_DEV - 1, N_DEV)

        bsem = pltpu.get_barrier_semaphore()
        for nbr in (left, right):
            pl.semaphore_signal(bsem, inc=1, device_id=(nbr,),
                                device_id_type=pl.DeviceIdType.MESH)
        pl.semaphore_wait(bsem, 2)

        cos_v = cos_ref[...]
        sin_v = sin_ref[...]
        even_v = even_ref[...]
        odd_v = odd_ref[...]

        def rot(t):
            tr = pltpu.roll(t, 1, 1) * odd_v - pltpu.roll(t, -1, 1) * even_v
            return t * cos_v + tr * sin_v

        for b in range(B):
            xb = x_ref[b]
            q = rot(jnp.dot(xb, wq_ref[...], preferred_element_type=jnp.float32))
            k = rot(jnp.dot(xb, wk_ref[...], preferred_element_type=jnp.float32))
            v = jnp.dot(xb, wv_ref[...], preferred_element_type=jnp.float32)
            ctxs = []
            for h in range(HQ_LOCAL):
                sl = slice(h * DH, (h + 1) * DH)
                qh, kh, vh = q[:, sl], k[:, sl], v[:, sl]
                s = lax.dot_general(qh, kh, (((1,), (1,)), ((), ())),
                                    preferred_element_type=jnp.float32) * SCALE
                m = jnp.max(s, axis=1, keepdims=True)
                w = jnp.exp(s - m)
                w = w / jnp.sum(w, axis=1, keepdims=True)
                ctxs.append(jnp.dot(w, vh, preferred_element_type=jnp.float32))
            ctx = jnp.concatenate(ctxs, axis=1)
            partial_ref[pl.ds(b * SQ, SQ), :] = jnp.dot(
                ctx, wo_ref[...], preferred_element_type=jnp.float32)

        sends = []

        def send(src_at, dst_buf, idx, sem_s, sem_r):
            rdma = pltpu.make_async_remote_copy(
                src_ref=src_at, dst_ref=dst_buf.at[idx],
                send_sem=sem_s.at[idx], recv_sem=sem_r.at[idx],
                device_id=(right,), device_id_type=pl.DeviceIdType.MESH)
            rdma.start()
            sends.append(rdma)

        def wait_recv(buf, idx, sem_s, sem_r):
            pltpu.make_async_remote_copy(
                src_ref=buf.at[idx], dst_ref=buf.at[idx],
                send_sem=sem_s.at[idx], recv_sem=sem_r.at[idx],
                device_id=(right,),
                device_id_type=pl.DeviceIdType.MESH).wait_recv()

        send(partial_ref.at[pl.ds(me * CHUNK, CHUNK), :], rs_buf, 0,
             rs_send, rs_recv)
        for s in range(STEPS):
            wait_recv(rs_buf, s, rs_send, rs_recv)
            c = lax.rem(me - (s + 1) + 2 * N_DEV, N_DEV)
            rs_buf[s, :, :] = rs_buf[s] + partial_ref[pl.ds(c * CHUNK, CHUNK), :]
            if s + 1 < STEPS:
                send(rs_buf.at[s], rs_buf, s + 1, rs_send, rs_recv)

        myc = lax.rem(me + 1, N_DEV)
        result_ref[pl.ds(myc * CHUNK, CHUNK), :] = rs_buf[STEPS - 1]

        send(rs_buf.at[STEPS - 1], ag_buf, 0, ag_send, ag_recv)
        for t in range(STEPS):
            wait_recv(ag_buf, t, ag_send, ag_recv)
            c = lax.rem(me - t + 2 * N_DEV, N_DEV)
            result_ref[pl.ds(c * CHUNK, CHUNK), :] = ag_buf[t]
            if t + 1 < STEPS:
                send(ag_buf.at[t], ag_buf, t + 1, ag_send, ag_recv)

        for rdma in sends:
            rdma.wait_send()

        out_ref[0] = result_ref[pl.ds(0, SQ), :]
        out_ref[1] = result_ref[pl.ds(SQ, SQ), :]

    return pl.pallas_call(
        body,
        out_shape=jax.ShapeDtypeStruct((B, SQ, D), jnp.float32),
        in_specs=[pl.BlockSpec(memory_space=pltpu.VMEM)] * 9,
        out_specs=pl.BlockSpec(memory_space=pltpu.VMEM),
        scratch_shapes=[
            pltpu.VMEM((ROWS, D), jnp.float32),
            pltpu.VMEM((ROWS, D), jnp.float32),
            pltpu.VMEM((STEPS, CHUNK, D), jnp.float32),
            pltpu.VMEM((STEPS, CHUNK, D), jnp.float32),
            pltpu.SemaphoreType.DMA((STEPS,)),
            pltpu.SemaphoreType.DMA((STEPS,)),
            pltpu.SemaphoreType.DMA((STEPS,)),
            pltpu.SemaphoreType.DMA((STEPS,)),
        ],
        compiler_params=pltpu.CompilerParams(collective_id=0),
    )(x, Wq, Wk, Wv, Wo, cos_t, sin_t, even, odd)


# baseline (device time: 109119 ns/iter reference)
import numpy as np
import jax
import jax.numpy as jnp
from jax import lax
from jax.experimental import pallas as pl
from jax.experimental.pallas import tpu as pltpu

N_DEV = 16
B, SQ, D = 2, 512, 1024
HQ_LOCAL, DH = 8, 128
SCALE = 0.08838834764831843
ROWS = B * SQ
CHUNK = ROWS // N_DEV
CW_STEPS = 8


def _rope_tables():
    inv = 1.0 / (10000.0 ** (np.arange(0, DH, 2) / DH))
    pos = np.arange(SQ)[:, None] * inv[None, :]
    cos = np.repeat(np.cos(pos), 2, axis=-1).astype(np.float32)
    sin = np.repeat(np.sin(pos), 2, axis=-1).astype(np.float32)
    cos_t = np.tile(cos, (1, HQ_LOCAL))
    sin_t = np.tile(sin, (1, HQ_LOCAL))
    even = np.tile(np.array([1.0, 0.0], np.float32), D // 2)[None, :]
    odd = 1.0 - even
    return cos_t, sin_t, even, odd


def kernel(x, Wq, Wk, Wv, Wo):
    cos_t, sin_t, even, odd = (jnp.asarray(a) for a in _rope_tables())

    def body(x_ref, wq_ref, wk_ref, wv_ref, wo_ref,
             cos_ref, sin_ref, even_ref, odd_ref, out_ref,
             partial_ref, result_ref, my_bf,
             cw_rs, ccw_rs, cw_ag, ccw_ag,
             cw_rs_s, cw_rs_r, ccw_rs_s, ccw_rs_r,
             cw_ag_s, cw_ag_r, ccw_ag_s, ccw_ag_r):
        me = lax.axis_index("i")
        right = lax.rem(me + 1, N_DEV)
        left = lax.rem(me + N_DEV - 1, N_DEV)

        def cidx(k):
            return lax.rem(me + k + 2 * N_DEV, N_DEV)

        bsem = pltpu.get_barrier_semaphore()
        for nbr in (left, right):
            pl.semaphore_signal(bsem, inc=1, device_id=(nbr,),
                                device_id_type=pl.DeviceIdType.MESH)
        pl.semaphore_wait(bsem, 2)

        cos_v = cos_ref[...]
        sin_v = sin_ref[...]
        even_v = even_ref[...]
        odd_v = odd_ref[...]

        def rot(t):
            tr = pltpu.roll(t, 1, 1) * odd_v - pltpu.roll(t, D - 1, 1) * even_v
            return t * cos_v + tr * sin_v

        for b in range(B):
            xb = x_ref[b]
            q = rot(jnp.dot(xb, wq_ref[...], preferred_element_type=jnp.float32))
            k = rot(jnp.dot(xb, wk_ref[...], preferred_element_type=jnp.float32))
            v = jnp.dot(xb, wv_ref[...], preferred_element_type=jnp.float32)
            ctxs = []
            for h in range(HQ_LOCAL):
                sl = slice(h * DH, (h + 1) * DH)
                qh, kh, vh = q[:, sl], k[:, sl], v[:, sl]
                s = lax.dot_general(qh, kh, (((1,), (1,)), ((), ())),
                                    preferred_element_type=jnp.float32) * SCALE
                m = jnp.max(s, axis=1, keepdims=True)
                w = jnp.exp(s - m)
                w = w / jnp.sum(w, axis=1, keepdims=True)
                ctxs.append(jnp.dot(w, vh, preferred_element_type=jnp.float32))
            ctx = jnp.concatenate(ctxs, axis=1)
            partial_ref[pl.ds(b * SQ, SQ), :] = jnp.dot(
                ctx, wo_ref[...], preferred_element_type=jnp.float32)

        sends = []

        def send(src_at, dst_buf, idx, sem_s, sem_r, dev):
            rdma = pltpu.make_async_remote_copy(
                src_ref=src_at, dst_ref=dst_buf.at[idx],
                send_sem=sem_s.at[idx], recv_sem=sem_r.at[idx],
                device_id=(dev,), device_id_type=pl.DeviceIdType.MESH)
            rdma.start()
            sends.append(rdma)

        def wait_recv(buf, idx, sem_s, sem_r):
            pltpu.make_async_remote_copy(
                src_ref=buf.at[idx], dst_ref=buf.at[idx],
                send_sem=sem_s.at[idx], recv_sem=sem_r.at[idx],
                device_id=(right,),
                device_id_type=pl.DeviceIdType.MESH).wait_recv()

        def pslice(k):
            return partial_ref.at[pl.ds(cidx(k) * CHUNK, CHUNK), :]

        send(pslice(8), cw_rs, 0, cw_rs_s, cw_rs_r, right)
        send(pslice(9), ccw_rs, 0, ccw_rs_s, ccw_rs_r, left)
        for s in range(CW_STEPS):
            wait_recv(cw_rs, s, cw_rs_s, cw_rs_r)
            if s < 7:
                cw_rs[s, :, :] = cw_rs[s] + partial_ref[
                    pl.ds(cidx(7 - s) * CHUNK, CHUNK), :]
                send(cw_rs.at[s], cw_rs, s + 1, cw_rs_s, cw_rs_r, right)
            if s < 7:
                wait_recv(ccw_rs, s, ccw_rs_s, ccw_rs_r)
                if s < 6:
                    ccw_rs[s, :, :] = ccw_rs[s] + partial_ref[
                        pl.ds(cidx(s - 6) * CHUNK, CHUNK), :]
                    send(ccw_rs.at[s], ccw_rs, s + 1, ccw_rs_s, ccw_rs_r, left)

        red = (cw_rs[7] + partial_ref[pl.ds(cidx(0) * CHUNK, CHUNK), :]
               + ccw_rs[6])
        result_ref[pl.ds(cidx(0) * CHUNK, CHUNK), :] = red
        my_bf[...] = red.astype(jnp.bfloat16)

        send(my_bf, cw_ag, 0, cw_ag_s, cw_ag_r, right)
        send(my_bf, ccw_ag, 0, ccw_ag_s, ccw_ag_r, left)
        for u in range(CW_STEPS):
            wait_recv(cw_ag, u, cw_ag_s, cw_ag_r)
            result_ref[pl.ds(cidx(-1 - u) * CHUNK, CHUNK), :] = (
                cw_ag[u].astype(jnp.float32))
            if u < 7:
                send(cw_ag.at[u], cw_ag, u + 1, cw_ag_s, cw_ag_r, right)
            if u < 7:
                wait_recv(ccw_ag, u, ccw_ag_s, ccw_ag_r)
                result_ref[pl.ds(cidx(1 + u) * CHUNK, CHUNK), :] = (
                    ccw_ag[u].astype(jnp.float32))
                if u < 6:
                    send(ccw_ag.at[u], ccw_ag, u + 1, ccw_ag_s, ccw_ag_r, left)

        for rdma in sends:
            rdma.wait_send()

        out_ref[0] = result_ref[pl.ds(0, SQ), :]
        out_ref[1] = result_ref[pl.ds(SQ, SQ), :]

    return pl.pallas_call(
        body,
        out_shape=jax.ShapeDtypeStruct((B, SQ, D), jnp.float32),
        in_specs=[pl.BlockSpec(memory_space=pltpu.VMEM)] * 9,
        out_specs=pl.BlockSpec(memory_space=pltpu.VMEM),
        scratch_shapes=[
            pltpu.VMEM((ROWS, D), jnp.float32),
            pltpu.VMEM((ROWS, D), jnp.float32),
            pltpu.VMEM((CHUNK, D), jnp.bfloat16),
            pltpu.VMEM((CW_STEPS, CHUNK, D), jnp.float32),
            pltpu.VMEM((CW_STEPS, CHUNK, D), jnp.float32),
            pltpu.VMEM((CW_STEPS, CHUNK, D), jnp.bfloat16),
            pltpu.VMEM((CW_STEPS, CHUNK, D), jnp.bfloat16),
            pltpu.SemaphoreType.DMA((CW_STEPS,)),
            pltpu.SemaphoreType.DMA((CW_STEPS,)),
            pltpu.SemaphoreType.DMA((CW_STEPS,)),
            pltpu.SemaphoreType.DMA((CW_STEPS,)),
            pltpu.SemaphoreType.DMA((CW_STEPS,)),
            pltpu.SemaphoreType.DMA((CW_STEPS,)),
            pltpu.SemaphoreType.DMA((CW_STEPS,)),
            pltpu.SemaphoreType.DMA((CW_STEPS,)),
        ],
        compiler_params=pltpu.CompilerParams(
            collective_id=0, vmem_limit_bytes=100 * 1024 * 1024),
    )(x, Wq, Wk, Wv, Wo, cos_t, sin_t, even, odd)


# device time: 98041 ns/iter; 1.1130x vs baseline; 1.1130x over previous
import numpy as np
import jax
import jax.numpy as jnp
from jax import lax
from jax.experimental import pallas as pl
from jax.experimental.pallas import tpu as pltpu

N_DEV = 16
B, SQ, D = 2, 512, 1024
HQ_LOCAL, DH = 8, 128
SCALE = 0.08838834764831843
ROWS = B * SQ
CHUNK = ROWS // N_DEV
CW_STEPS = 8


def _rope_tables():
    inv = 1.0 / (10000.0 ** (np.arange(0, DH, 2) / DH))
    pos = np.arange(SQ)[:, None] * inv[None, :]
    cos = np.repeat(np.cos(pos), 2, axis=-1).astype(np.float32)
    sin = np.repeat(np.sin(pos), 2, axis=-1).astype(np.float32)
    cos_t = np.tile(cos, (1, HQ_LOCAL))
    sin_t = np.tile(sin, (1, HQ_LOCAL))
    even = np.tile(np.array([1.0, 0.0], np.float32), D // 2)[None, :]
    odd = 1.0 - even
    return cos_t, sin_t, even, odd


def kernel(x, Wq, Wk, Wv, Wo):
    cos_t, sin_t, even, odd = (jnp.asarray(a) for a in _rope_tables())

    def body(x_ref, wq_ref, wk_ref, wv_ref, wo_ref,
             cos_ref, sin_ref, even_ref, odd_ref, out_ref,
             partial_ref, result_ref, my_bf, seed_bf,
             cw_rs, ccw_rs, cw_ag, ccw_ag,
             cw_rs_s, cw_rs_r, ccw_rs_s, ccw_rs_r,
             cw_ag_s, cw_ag_r, ccw_ag_s, ccw_ag_r):
        me = lax.axis_index("i")
        right = lax.rem(me + 1, N_DEV)
        left = lax.rem(me + N_DEV - 1, N_DEV)

        def cidx(k):
            return lax.rem(me + k + 2 * N_DEV, N_DEV)

        bsem = pltpu.get_barrier_semaphore()
        for nbr in (left, right):
            pl.semaphore_signal(bsem, inc=1, device_id=(nbr,),
                                device_id_type=pl.DeviceIdType.MESH)
        pl.semaphore_wait(bsem, 2)

        cos_v = cos_ref[...]
        sin_v = sin_ref[...]
        even_v = even_ref[...]
        odd_v = odd_ref[...]

        def rot(t):
            tr = pltpu.roll(t, 1, 1) * odd_v - pltpu.roll(t, D - 1, 1) * even_v
            return t * cos_v + tr * sin_v

        for b in range(B):
            xb = x_ref[b]
            q = rot(jnp.dot(xb, wq_ref[...], preferred_element_type=jnp.float32))
            k = rot(jnp.dot(xb, wk_ref[...], preferred_element_type=jnp.float32))
            v = jnp.dot(xb, wv_ref[...], preferred_element_type=jnp.float32)
            ctxs = []
            for h in range(HQ_LOCAL):
                sl = slice(h * DH, (h + 1) * DH)
                qh, kh, vh = q[:, sl], k[:, sl], v[:, sl]
                s = lax.dot_general(qh, kh, (((1,), (1,)), ((), ())),
                                    preferred_element_type=jnp.float32) * SCALE
                m = jnp.max(s, axis=1, keepdims=True)
                w = jnp.exp(s - m)
                w = w / jnp.sum(w, axis=1, keepdims=True)
                ctxs.append(jnp.dot(w, vh, preferred_element_type=jnp.float32))
            ctx = jnp.concatenate(ctxs, axis=1)
            partial_ref[pl.ds(b * SQ, SQ), :] = jnp.dot(
                ctx, wo_ref[...], preferred_element_type=jnp.float32)

        sends = []

        def send(src_at, dst_buf, idx, sem_s, sem_r, dev):
            rdma = pltpu.make_async_remote_copy(
                src_ref=src_at, dst_ref=dst_buf.at[idx],
                send_sem=sem_s.at[idx], recv_sem=sem_r.at[idx],
                device_id=(dev,), device_id_type=pl.DeviceIdType.MESH)
            rdma.start()
            sends.append(rdma)

        def wait_recv(buf, idx, sem_s, sem_r):
            pltpu.make_async_remote_copy(
                src_ref=buf.at[idx], dst_ref=buf.at[idx],
                send_sem=sem_s.at[idx], recv_sem=sem_r.at[idx],
                device_id=(right,),
                device_id_type=pl.DeviceIdType.MESH).wait_recv()

        def pchunk(k):
            return partial_ref[pl.ds(cidx(k) * CHUNK, CHUNK), :]

        seed_bf[0, :, :] = pchunk(8).astype(jnp.bfloat16)
        seed_bf[1, :, :] = pchunk(9).astype(jnp.bfloat16)
        send(seed_bf.at[0], cw_rs, 0, cw_rs_s, cw_rs_r, right)
        send(seed_bf.at[1], ccw_rs, 0, ccw_rs_s, ccw_rs_r, left)
        for s in range(CW_STEPS):
            wait_recv(cw_rs, s, cw_rs_s, cw_rs_r)
            if s < 7:
                cw_rs[s, :, :] = (cw_rs[s].astype(jnp.float32)
                                  + pchunk(7 - s)).astype(jnp.bfloat16)
                send(cw_rs.at[s], cw_rs, s + 1, cw_rs_s, cw_rs_r, right)
            if s < 7:
                wait_recv(ccw_rs, s, ccw_rs_s, ccw_rs_r)
                if s < 6:
                    ccw_rs[s, :, :] = (ccw_rs[s].astype(jnp.float32)
                                       + pchunk(s - 6)).astype(jnp.bfloat16)
                    send(ccw_rs.at[s], ccw_rs, s + 1, ccw_rs_s, ccw_rs_r, left)

        red = (cw_rs[7].astype(jnp.float32) + pchunk(0)
               + ccw_rs[6].astype(jnp.float32))
        result_ref[pl.ds(cidx(0) * CHUNK, CHUNK), :] = red
        my_bf[...] = red.astype(jnp.bfloat16)

        send(my_bf, cw_ag, 0, cw_ag_s, cw_ag_r, right)
        send(my_bf, ccw_ag, 0, ccw_ag_s, ccw_ag_r, left)
        for u in range(CW_STEPS):
            wait_recv(cw_ag, u, cw_ag_s, cw_ag_r)
            result_ref[pl.ds(cidx(-1 - u) * CHUNK, CHUNK), :] = (
                cw_ag[u].astype(jnp.float32))
            if u < 7:
                send(cw_ag.at[u], cw_ag, u + 1, cw_ag_s, cw_ag_r, right)
            if u < 7:
                wait_recv(ccw_ag, u, ccw_ag_s, ccw_ag_r)
                result_ref[pl.ds(cidx(1 + u) * CHUNK, CHUNK), :] = (
                    ccw_ag[u].astype(jnp.float32))
                if u < 6:
                    send(ccw_ag.at[u], ccw_ag, u + 1, ccw_ag_s, ccw_ag_r, left)

        for rdma in sends:
            rdma.wait_send()

        out_ref[0] = result_ref[pl.ds(0, SQ), :]
        out_ref[1] = result_ref[pl.ds(SQ, SQ), :]

    return pl.pallas_call(
        body,
        out_shape=jax.ShapeDtypeStruct((B, SQ, D), jnp.float32),
        in_specs=[pl.BlockSpec(memory_space=pltpu.VMEM)] * 9,
        out_specs=pl.BlockSpec(memory_space=pltpu.VMEM),
        scratch_shapes=[
            pltpu.VMEM((ROWS, D), jnp.float32),
            pltpu.VMEM((ROWS, D), jnp.float32),
            pltpu.VMEM((CHUNK, D), jnp.bfloat16),
            pltpu.VMEM((2, CHUNK, D), jnp.bfloat16),
            pltpu.VMEM((CW_STEPS, CHUNK, D), jnp.bfloat16),
            pltpu.VMEM((CW_STEPS, CHUNK, D), jnp.bfloat16),
            pltpu.VMEM((CW_STEPS, CHUNK, D), jnp.bfloat16),
            pltpu.VMEM((CW_STEPS, CHUNK, D), jnp.bfloat16),
            pltpu.SemaphoreType.DMA((CW_STEPS,)),
            pltpu.SemaphoreType.DMA((CW_STEPS,)),
            pltpu.SemaphoreType.DMA((CW_STEPS,)),
            pltpu.SemaphoreType.DMA((CW_STEPS,)),
            pltpu.SemaphoreType.DMA((CW_STEPS,)),
            pltpu.SemaphoreType.DMA((CW_STEPS,)),
            pltpu.SemaphoreType.DMA((CW_STEPS,)),
            pltpu.SemaphoreType.DMA((CW_STEPS,)),
        ],
        compiler_params=pltpu.CompilerParams(
            collective_id=0, vmem_limit_bytes=100 * 1024 * 1024),
    )(x, Wq, Wk, Wv, Wo, cos_t, sin_t, even, odd)
